# baseline (speedup 1.0000x reference)
import jax
import jax.numpy as jnp
from jax.experimental import pallas as pl
from jax.experimental.pallas import tpu as pltpu

N = 4096
D = 256
CH = 256
NBUF = 3
STEPS = N // CH


def _body(adj_hbm, emb_ref, out_ref, bufs, sems):
    def start(i, slot):
        pltpu.make_async_copy(
            adj_hbm.at[pl.ds(i * CH, CH), :], bufs.at[slot], sems.at[slot]
        ).start()

    def wait(slot):
        pltpu.make_async_copy(
            adj_hbm.at[pl.ds(0, CH), :], bufs.at[slot], sems.at[slot]
        ).wait()

    for s in range(NBUF):
        start(s, s)
    for i in range(STEPS):
        slot = i % NBUF
        wait(slot)
        out_ref[pl.ds(i * CH, CH), :] = jnp.dot(
            bufs[slot], emb_ref[...], preferred_element_type=jnp.float32
        )
        if i + NBUF < STEPS:
            start(i + NBUF, slot)


@jax.jit
def kernel(adj, embeds):
    return pl.pallas_call(
        _body,
        in_specs=[
            pl.BlockSpec(memory_space=pltpu.MemorySpace.HBM),
            pl.BlockSpec((N, D), lambda: (0, 0)),
        ],
        out_specs=pl.BlockSpec((N, D), lambda: (0, 0)),
        out_shape=jax.ShapeDtypeStruct((N, D), jnp.float32),
        scratch_shapes=[
            pltpu.VMEM((NBUF, CH, N), jnp.float32),
            pltpu.SemaphoreType.DMA((NBUF,)),
        ],
    )(adj, embeds)


# speedup vs baseline: 1.0573x; 1.0573x over previous
import jax
import jax.numpy as jnp
from jax.experimental import pallas as pl
from jax.experimental.pallas import tpu as pltpu

N = 4096
D = 256
S = 2
BR = 256


def _body(*refs):
    adj_refs = refs[:S]
    emb_ref = refs[S]
    out_ref = refs[S + 1]
    for s in range(S):
        out_ref[s] = jnp.dot(
            adj_refs[s][0], emb_ref[...], preferred_element_type=jnp.float32
        )


@jax.jit
def kernel(adj, embeds):
    adj3 = adj.reshape(S, N // S, N)
    grid = (N // S // BR,)
    in_specs = [
        pl.BlockSpec((1, BR, N), (lambda i, s=s: (s, i, 0))) for s in range(S)
    ] + [pl.BlockSpec((N, D), lambda i: (0, 0))]
    out = pl.pallas_call(
        _body,
        grid=grid,
        in_specs=in_specs,
        out_specs=pl.BlockSpec((S, BR, D), lambda i: (0, i, 0)),
        out_shape=jax.ShapeDtypeStruct((S, N // S, D), jnp.float32),
        compiler_params=pltpu.CompilerParams(
            dimension_semantics=("arbitrary",),
        ),
    )(*([adj3] * S), embeds)
    return out.reshape(N, D)
